# baseline (speedup 1.0000x reference)
import functools

import jax
import jax.numpy as jnp
from jax import lax
from jax.experimental import pallas as pl
from jax.experimental.pallas import tpu as pltpu
from jax.experimental.pallas import tpu_sc as plsc

_N = 256
_DT = 0.05
_LANES = 16
_GROUPS = 2


def _make_body(nc, per_w, n_chunks):
    chunk_span = 2 * _LANES
    chunks_per_group = n_chunks // _GROUPS
    group_words = per_w // _GROUPS

    def body(x_hbm, f_hbm, o_hbm, buf, tab, sem_t, sem_x, sem_o):
        wid = lax.axis_index("s") * nc + lax.axis_index("c")
        base = wid * per_w
        h_t = pltpu.async_copy(f_hbm, tab, sem_t)
        ins = [
            pltpu.async_copy(
                x_hbm.at[pl.ds(base + g * group_words, group_words)],
                buf.at[pl.ds(g * group_words, group_words)],
                sem_x,
            )
            for g in range(_GROUPS)
        ]
        h_t.wait()
        odd = lax.iota(jnp.int32, _LANES) * 2 + 1
        outs = []
        for g in range(_GROUPS):
            ins[g].wait()

            @plsc.parallel_loop(
                g * chunks_per_group, (g + 1) * chunks_per_group, unroll=4
            )
            def chunk(i):
                idx = odd + (i * chunk_span)
                w = plsc.load_gather(buf, [idx])
                fi = w.astype(jnp.int32)
                a = w - fi.astype(jnp.float32)
                ci = jnp.minimum(fi + 1, _N)
                f0 = plsc.load_gather(tab, [fi])
                f1 = plsc.load_gather(tab, [ci])
                stepped = w + _DT * (f0 + a * (f1 - f0))
                plsc.store_scatter(buf, [idx], stepped)
            outs.append(
                pltpu.async_copy(
                    buf.at[pl.ds(g * group_words, group_words)],
                    o_hbm.at[pl.ds(base + g * group_words, group_words)],
                    sem_o,
                )
            )
        for h in outs:
            h.wait()

    return body


@functools.lru_cache(maxsize=None)
def _build(total_words, tab_words):
    info = plsc.get_sparse_core_info()
    nc, ns = info.num_cores, info.num_subcores
    nw = nc * ns
    per_w = total_words // nw
    n_chunks = per_w // (2 * _LANES)
    mesh = plsc.VectorSubcoreMesh(
        core_axis_name="c", subcore_axis_name="s", num_cores=nc
    )
    return pl.kernel(
        _make_body(nc, per_w, n_chunks),
        mesh=mesh,
        out_type=jax.ShapeDtypeStruct((total_words,), jnp.float32),
        compiler_params=pltpu.CompilerParams(needs_layout_passes=False),
        scratch_types=[
            pltpu.VMEM((per_w,), jnp.float32),
            pltpu.VMEM((tab_words,), jnp.float32),
            pltpu.SemaphoreType.DMA,
            pltpu.SemaphoreType.DMA,
            pltpu.SemaphoreType.DMA,
        ],
    )


def kernel(X, force):
    rows = X.shape[0]
    flat = X.reshape(-1)
    out = _build(flat.shape[0], force.shape[0])(flat, force)
    return out.reshape(rows, 2)


# speedup vs baseline: 1.0022x; 1.0022x over previous
import functools

import jax
import jax.numpy as jnp
from jax import lax
from jax.experimental import pallas as pl
from jax.experimental.pallas import tpu as pltpu
from jax.experimental.pallas import tpu_sc as plsc

_N = 256
_DT = 0.05
_LANES = 16
_GROUPS = 1


def _make_body(nc, per_w, n_chunks):
    chunk_span = 2 * _LANES
    chunks_per_group = n_chunks // _GROUPS
    group_words = per_w // _GROUPS

    def body(x_hbm, f_hbm, o_hbm, buf, tab, sem_t, sem_x, sem_o):
        wid = lax.axis_index("s") * nc + lax.axis_index("c")
        base = wid * per_w
        h_t = pltpu.async_copy(f_hbm, tab, sem_t)
        ins = [
            pltpu.async_copy(
                x_hbm.at[pl.ds(base + g * group_words, group_words)],
                buf.at[pl.ds(g * group_words, group_words)],
                sem_x,
            )
            for g in range(_GROUPS)
        ]
        h_t.wait()
        odd = lax.iota(jnp.int32, _LANES) * 2 + 1
        outs = []
        for g in range(_GROUPS):
            ins[g].wait()

            @plsc.parallel_loop(
                g * chunks_per_group, (g + 1) * chunks_per_group, unroll=8
            )
            def chunk(i):
                idx = odd + (i * chunk_span)
                w = plsc.load_gather(buf, [idx])
                fi = w.astype(jnp.int32)
                a = w - fi.astype(jnp.float32)
                ci = jnp.minimum(fi + 1, _N)
                f0 = plsc.load_gather(tab, [fi])
                f1 = plsc.load_gather(tab, [ci])
                stepped = w + _DT * (f0 + a * (f1 - f0))
                plsc.store_scatter(buf, [idx], stepped)
            outs.append(
                pltpu.async_copy(
                    buf.at[pl.ds(g * group_words, group_words)],
                    o_hbm.at[pl.ds(base + g * group_words, group_words)],
                    sem_o,
                )
            )
        for h in outs:
            h.wait()

    return body


@functools.lru_cache(maxsize=None)
def _build(total_words, tab_words):
    info = plsc.get_sparse_core_info()
    nc, ns = info.num_cores, info.num_subcores
    nw = nc * ns
    per_w = total_words // nw
    n_chunks = per_w // (2 * _LANES)
    mesh = plsc.VectorSubcoreMesh(
        core_axis_name="c", subcore_axis_name="s", num_cores=nc
    )
    return pl.kernel(
        _make_body(nc, per_w, n_chunks),
        mesh=mesh,
        out_type=jax.ShapeDtypeStruct((total_words,), jnp.float32),
        compiler_params=pltpu.CompilerParams(needs_layout_passes=False),
        scratch_types=[
            pltpu.VMEM((per_w,), jnp.float32),
            pltpu.VMEM((tab_words,), jnp.float32),
            pltpu.SemaphoreType.DMA,
            pltpu.SemaphoreType.DMA,
            pltpu.SemaphoreType.DMA,
        ],
    )


def kernel(X, force):
    rows = X.shape[0]
    flat = X.reshape(-1)
    out = _build(flat.shape[0], force.shape[0])(flat, force)
    return out.reshape(rows, 2)


# speedup vs baseline: 1.0032x; 1.0010x over previous
import functools

import jax
import jax.numpy as jnp
from jax import lax
from jax.experimental import pallas as pl
from jax.experimental.pallas import tpu as pltpu
from jax.experimental.pallas import tpu_sc as plsc

_N = 256
_DT = 0.05
_LANES = 16
_GROUPS = 1


def _make_body(nc, per_w, n_chunks):
    chunk_span = 2 * _LANES
    chunks_per_group = n_chunks // _GROUPS
    group_words = per_w // _GROUPS

    def body(x_hbm, f_hbm, o_hbm, buf, tab, sem_t, sem_x, sem_o):
        wid = lax.axis_index("s") * nc + lax.axis_index("c")
        base = wid * per_w
        h_t = pltpu.async_copy(f_hbm, tab, sem_t)
        ins = [
            pltpu.async_copy(
                x_hbm.at[pl.ds(base + g * group_words, group_words)],
                buf.at[pl.ds(g * group_words, group_words)],
                sem_x,
            )
            for g in range(_GROUPS)
        ]
        h_t.wait()
        odd = lax.iota(jnp.int32, _LANES) * 2 + 1
        outs = []
        for g in range(_GROUPS):
            ins[g].wait()

            @plsc.parallel_loop(
                g * chunks_per_group, (g + 1) * chunks_per_group, unroll=4
            )
            def chunk(i):
                idx = odd + (i * chunk_span)
                w = plsc.load_gather(buf, [idx])
                fi = w.astype(jnp.int32)
                a = w - fi.astype(jnp.float32)
                ci = jnp.minimum(fi + 1, _N)
                f0 = plsc.load_gather(tab, [fi])
                f1 = plsc.load_gather(tab, [ci])
                stepped = w + _DT * (f0 + a * (f1 - f0))
                plsc.store_scatter(buf, [idx], stepped)
            outs.append(
                pltpu.async_copy(
                    buf.at[pl.ds(g * group_words, group_words)],
                    o_hbm.at[pl.ds(base + g * group_words, group_words)],
                    sem_o,
                )
            )
        for h in outs:
            h.wait()

    return body


@functools.lru_cache(maxsize=None)
def _build(total_words, tab_words):
    info = plsc.get_sparse_core_info()
    nc, ns = info.num_cores, info.num_subcores
    nw = nc * ns
    per_w = total_words // nw
    n_chunks = per_w // (2 * _LANES)
    mesh = plsc.VectorSubcoreMesh(
        core_axis_name="c", subcore_axis_name="s", num_cores=nc
    )
    return pl.kernel(
        _make_body(nc, per_w, n_chunks),
        mesh=mesh,
        out_type=jax.ShapeDtypeStruct((total_words,), jnp.float32),
        compiler_params=pltpu.CompilerParams(needs_layout_passes=False),
        scratch_types=[
            pltpu.VMEM((per_w,), jnp.float32),
            pltpu.VMEM((tab_words,), jnp.float32),
            pltpu.SemaphoreType.DMA,
            pltpu.SemaphoreType.DMA,
            pltpu.SemaphoreType.DMA,
        ],
    )


def kernel(X, force):
    rows = X.shape[0]
    flat = X.reshape(-1)
    out = _build(flat.shape[0], force.shape[0])(flat, force)
    return out.reshape(rows, 2)


# speedup vs baseline: 1.3732x; 1.3689x over previous
import functools

import jax
import jax.numpy as jnp
from jax import lax
from jax.experimental import pallas as pl
from jax.experimental.pallas import tpu as pltpu
from jax.experimental.pallas import tpu_sc as plsc

_N = 256
_DT = 0.05
_LANES = 16


def _make_body(nc, rows_w, n_chunks):
    def body(x_hbm, f_hbm, o_hbm, buf, tab, sem_t, sem_x, sem_o):
        wid = lax.axis_index("s") * nc + lax.axis_index("c")
        base = wid * rows_w
        groups = 4
        rows_g = rows_w // groups
        chunks_g = n_chunks // groups
        h_t = pltpu.async_copy(f_hbm, tab, sem_t)
        ins = [
            pltpu.async_copy(
                x_hbm.at[pl.ds(base + g * rows_g, rows_g)],
                buf.at[pl.ds(g * rows_g, rows_g)],
                sem_x,
            )
            for g in range(groups)
        ]
        h_t.wait()
        lane = lax.iota(jnp.int32, _LANES)
        vcol = jnp.full((_LANES,), 1, jnp.int32)
        outs = []
        for g in range(groups):
            ins[g].wait()

            @plsc.parallel_loop(g * chunks_g, (g + 1) * chunks_g, unroll=4)
            def chunk(i):
                ridx = lane + i * _LANES
                w = plsc.load_gather(buf, [ridx, vcol])
                fi = w.astype(jnp.int32)
                a = w - fi.astype(jnp.float32)
                ci = jnp.minimum(fi + 1, _N)
                f0 = plsc.load_gather(tab, [fi])
                f1 = plsc.load_gather(tab, [ci])
                stepped = w + _DT * (f0 + a * (f1 - f0))
                plsc.store_scatter(buf, [ridx, vcol], stepped)

            outs.append(
                pltpu.async_copy(
                    buf.at[pl.ds(g * rows_g, rows_g)],
                    o_hbm.at[pl.ds(base + g * rows_g, rows_g)],
                    sem_o,
                )
            )
        for h in outs:
            h.wait()

    return body


@functools.lru_cache(maxsize=None)
def _build(rows, tab_words):
    info = plsc.get_sparse_core_info()
    nc, ns = info.num_cores, info.num_subcores
    nw = nc * ns
    rows_w = rows // nw
    n_chunks = rows_w // _LANES
    mesh = plsc.VectorSubcoreMesh(
        core_axis_name="c", subcore_axis_name="s", num_cores=nc
    )
    return pl.kernel(
        _make_body(nc, rows_w, n_chunks),
        mesh=mesh,
        out_type=jax.ShapeDtypeStruct((rows, 2), jnp.float32),
        compiler_params=pltpu.CompilerParams(needs_layout_passes=False),
        scratch_types=[
            pltpu.VMEM((rows_w, 2), jnp.float32),
            pltpu.VMEM((tab_words,), jnp.float32),
            pltpu.SemaphoreType.DMA,
            pltpu.SemaphoreType.DMA,
            pltpu.SemaphoreType.DMA,
        ],
    )


def kernel(X, force):
    return _build(X.shape[0], force.shape[0])(X, force)
